# baseline (speedup 1.0000x reference)
import functools

import jax
import jax.numpy as jnp
from jax.experimental import pallas as pl
from jax.experimental.pallas import tpu as pltpu

_F32 = jnp.float32

_EF4 = ((1.0, 0.0), (-1.0, 0.0), (0.0, 1.0), (0.0, -1.0))


def _dot(a, b):
    return jnp.dot(a, b, preferred_element_type=_F32)


def _gru_core(nx, h, hid, w4, nnb, gwih, gbih, gwhh, gbhh):
    n = hid.shape[0]
    row = jax.lax.broadcasted_iota(jnp.int32, (n, 1), 0)
    j = row % nx
    i = (row // nx) % nx
    m0 = (j >= 1)
    m1 = (j <= nx - 2)
    m2 = (i >= 1)
    m3 = (i <= nx - 2)
    z1 = jnp.zeros((1, h), _F32)
    znx = jnp.zeros((nx, h), _F32)
    s0 = jnp.where(m0, jnp.concatenate([z1, hid[:-1]], axis=0), 0.0)
    s1 = jnp.where(m1, jnp.concatenate([hid[1:], z1], axis=0), 0.0)
    s2 = jnp.where(m2, jnp.concatenate([znx, hid[:-nx]], axis=0), 0.0)
    s3 = jnp.where(m3, jnp.concatenate([hid[nx:], znx], axis=0), 0.0)
    xcat = jnp.concatenate([s0, s1, s2, s3], axis=1)
    ssum = _dot(xcat, w4)
    deg = (m0.astype(_F32) + m1.astype(_F32) + m2.astype(_F32)
           + m3.astype(_F32))
    m = jnp.maximum(ssum * (1.0 / deg) + nnb, 0.0)
    gi = _dot(m, gwih) + gbih
    gh = _dot(hid, gwhh) + gbhh
    r = jax.nn.sigmoid(gi[:, :h] + gh[:, :h])
    z = jax.nn.sigmoid(gi[:, h:2 * h] + gh[:, h:2 * h])
    nn = jnp.tanh(gi[:, 2 * h:] + r * gh[:, 2 * h:])
    return (1.0 - z) * nn + z * hid


def _unet_kern(t, nx, h1, h2,
               x_ref,
               aw1, ab1, aw2, ab2, a4, anb, awih, abih, awhh, abhh,
               bw1, bb1, bw2, bb2, b4, bnb, bwih, bbih, bwhh, bbhh,
               cwa, cwb, cb1, cw2, cb2, c4, cnb, cwih, cbih, cwhh, cbhh,
               o_ref, spool_ref, sup_ref):
    nf = t * nx * nx
    nh = nx // 2
    nc = t * nh * nh

    l1 = jnp.maximum(_dot(x_ref[...], aw1[...]) + ab1[...], 0.0)
    hid = _dot(l1, aw2[...]) + ab2[...]
    bp = _gru_core(nx, h1, hid, a4[...], anb[...], awih[...],
                   abih[...], awhh[...], abhh[...])

    z1 = jnp.zeros((1, h1), _F32)
    spool_ref[...] = bp + jnp.concatenate([bp[1:], z1], axis=0)
    t1 = spool_ref[pl.Slice(0, nf // 2, 2), :]
    t4 = t1.reshape(t * nx // 2, 2, nh, h1)
    d = ((t4[:, 0] + t4[:, 1]) * 0.25).reshape(nc, h1)

    l1b = jnp.maximum(_dot(d, bw1[...]) + bb1[...], 0.0)
    hidb = _dot(l1b, bw2[...]) + bb2[...]
    h2v = _gru_core(nh, h2, hidb, b4[...], bnb[...],
                    bwih[...], bbih[...], bwhh[...], bbhh[...])

    u3 = h2v.reshape(t * nh, 1, nh, h2)
    ui = jnp.concatenate([u3, u3], axis=1).reshape(nf // 2, h2)
    v = _dot(ui, cwb[...])
    sup_ref[pl.Slice(0, nf // 2, 2), :] = v
    sup_ref[pl.Slice(1, nf // 2, 2), :] = v

    pre = _dot(bp, cwa[...]) + sup_ref[...] + cb1[...]
    l1c = jnp.maximum(pre, 0.0)
    hidc = _dot(l1c, cw2[...]) + cb2[...]
    o_ref[...] = _gru_core(nx, h1, hidc, c4[...], cnb[...],
                           cwih[...], cbih[...], cwhh[...], cbhh[...])


def _edge_w4(p, h):
    ef = jnp.asarray(_EF4, _F32)
    a = jnp.maximum(ef @ p['eW1'] + p['eb1'], 0.0)
    w = (a @ p['eW2'] + p['eb2']).reshape(4, h, h)
    return w.reshape(4 * h, h)


def _row(v):
    return v.reshape(1, -1)


def _block_args(p, h):
    return (p['pW1'], _row(p['pb1']), p['pW2'], _row(p['pb2']),
            _edge_w4(p, h), _row(p['nnb']), p['gWih'], _row(p['gbih']),
            p['gWhh'], _row(p['gbhh']))


def kernel(inputs, params):
    b, t, nx, ny, c = inputs.shape
    h1 = params['c1']['pb2'].shape[0]
    h2 = params['lw']['pb2'].shape[0]
    p2 = params['c2']
    cwa = p2['pW1'][:h1]
    cwb = params['upW'] @ p2['pW1'][h1:]
    cb1 = p2['pb1'] + params['upb'] @ p2['pW1'][h1:]
    c2_args = (cwa, cwb, _row(cb1), p2['pW2'], _row(p2['pb2']),
               _edge_w4(p2, h1), _row(p2['nnb']), p2['gWih'],
               _row(p2['gbih']), p2['gWhh'], _row(p2['gbhh']))
    gsplit = 2
    th = t // gsplit
    fn = functools.partial(_unet_kern, th, nx, h1, h2)
    nf = t * nx * ny
    nfh = nf // gsplit
    def _wspec(a):
        return pl.BlockSpec(a.shape, lambda g: (0,) * a.ndim)
    call = lambda xx, *ws: pl.pallas_call(
        fn,
        grid=(gsplit,),
        in_specs=[pl.BlockSpec((nfh, c), lambda g: (g, 0))]
        + [_wspec(w) for w in ws],
        out_specs=pl.BlockSpec((nfh, h1), lambda g: (g, 0)),
        out_shape=jax.ShapeDtypeStruct((nf, h1), _F32),
        scratch_shapes=[pltpu.VMEM((nfh, h1), _F32),
                        pltpu.VMEM((nfh, h1), _F32)],
        compiler_params=pltpu.CompilerParams(
            dimension_semantics=("parallel",)),
    )(xx, *ws)
    outs = []
    for bi in range(b):
        x = inputs[bi].reshape(t * nx * ny, c)
        h3 = call(x, *_block_args(params['c1'], h1),
                  *_block_args(params['lw'], h2), *c2_args)
        outs.append(h3.reshape(t, nx, ny, h1))
    return jnp.stack(outs, 0)


# speedup vs baseline: 1.0038x; 1.0038x over previous
import functools

import jax
import jax.numpy as jnp
from jax.experimental import pallas as pl
from jax.experimental.pallas import tpu as pltpu

_F32 = jnp.float32

_EF4 = ((1.0, 0.0), (-1.0, 0.0), (0.0, 1.0), (0.0, -1.0))


def _dot(a, b):
    return jnp.dot(a, b, preferred_element_type=_F32)


def _gru_core(nx, h, hid, w4, nnb, gwih, gbih, gwhh, gbhh):
    n = hid.shape[0]
    row = jax.lax.broadcasted_iota(jnp.int32, (n, 1), 0)
    j = row % nx
    i = (row // nx) % nx
    m0 = (j >= 1)
    m1 = (j <= nx - 2)
    m2 = (i >= 1)
    m3 = (i <= nx - 2)
    z1 = jnp.zeros((1, h), _F32)
    znx = jnp.zeros((nx, h), _F32)
    s0 = jnp.where(m0, jnp.concatenate([z1, hid[:-1]], axis=0), 0.0)
    s1 = jnp.where(m1, jnp.concatenate([hid[1:], z1], axis=0), 0.0)
    s2 = jnp.where(m2, jnp.concatenate([znx, hid[:-nx]], axis=0), 0.0)
    s3 = jnp.where(m3, jnp.concatenate([hid[nx:], znx], axis=0), 0.0)
    xcat = jnp.concatenate([s0, s1, s2, s3], axis=1)
    ssum = _dot(xcat, w4)
    deg = (m0.astype(_F32) + m1.astype(_F32) + m2.astype(_F32)
           + m3.astype(_F32))
    m = jnp.maximum(ssum * (1.0 / deg) + nnb, 0.0)
    gi = _dot(m, gwih) + gbih
    gh = _dot(hid, gwhh) + gbhh
    r = jax.nn.sigmoid(gi[:, :h] + gh[:, :h])
    z = jax.nn.sigmoid(gi[:, h:2 * h] + gh[:, h:2 * h])
    nn = jnp.tanh(gi[:, 2 * h:] + r * gh[:, 2 * h:])
    return (1.0 - z) * nn + z * hid


def _unet_kern(t, nx, h1, h2,
               x_ref,
               aw1, ab1, aw2, ab2, a4, anb, awih, abih, awhh, abhh,
               bw1, bb1, bw2, bb2, b4, bnb, bwih, bbih, bwhh, bbhh,
               cwa, cwb, cb1, cw2, cb2, c4, cnb, cwih, cbih, cwhh, cbhh,
               o_ref, spool_ref, sup_ref):
    nf = t * nx * nx
    nh = nx // 2
    nc = t * nh * nh

    x = x_ref[...].reshape(nf, x_ref.shape[-1])
    l1 = jnp.maximum(_dot(x, aw1[...]) + ab1[...], 0.0)
    hid = _dot(l1, aw2[...]) + ab2[...]
    bp = _gru_core(nx, h1, hid, a4[...], anb[...], awih[...],
                   abih[...], awhh[...], abhh[...])

    z1 = jnp.zeros((1, h1), _F32)
    spool_ref[...] = bp + jnp.concatenate([bp[1:], z1], axis=0)
    t1 = spool_ref[pl.Slice(0, nf // 2, 2), :]
    t4 = t1.reshape(t * nx // 2, 2, nh, h1)
    d = ((t4[:, 0] + t4[:, 1]) * 0.25).reshape(nc, h1)

    l1b = jnp.maximum(_dot(d, bw1[...]) + bb1[...], 0.0)
    hidb = _dot(l1b, bw2[...]) + bb2[...]
    h2v = _gru_core(nh, h2, hidb, b4[...], bnb[...],
                    bwih[...], bbih[...], bwhh[...], bbhh[...])

    u3 = h2v.reshape(t * nh, 1, nh, h2)
    ui = jnp.concatenate([u3, u3], axis=1).reshape(nf // 2, h2)
    v = _dot(ui, cwb[...])
    sup_ref[pl.Slice(0, nf // 2, 2), :] = v
    sup_ref[pl.Slice(1, nf // 2, 2), :] = v

    pre = _dot(bp, cwa[...]) + sup_ref[...] + cb1[...]
    l1c = jnp.maximum(pre, 0.0)
    hidc = _dot(l1c, cw2[...]) + cb2[...]
    res = _gru_core(nx, h1, hidc, c4[...], cnb[...],
                    cwih[...], cbih[...], cwhh[...], cbhh[...])
    o_ref[...] = res.reshape(o_ref.shape)


def _edge_w4(p, h):
    ef = jnp.asarray(_EF4, _F32)
    a = jnp.maximum(ef @ p['eW1'] + p['eb1'], 0.0)
    w = (a @ p['eW2'] + p['eb2']).reshape(4, h, h)
    return w.reshape(4 * h, h)


def _row(v):
    return v.reshape(1, -1)


def _block_args(p, h):
    return (p['pW1'], _row(p['pb1']), p['pW2'], _row(p['pb2']),
            _edge_w4(p, h), _row(p['nnb']), p['gWih'], _row(p['gbih']),
            p['gWhh'], _row(p['gbhh']))


def kernel(inputs, params):
    b, t, nx, ny, c = inputs.shape
    h1 = params['c1']['pb2'].shape[0]
    h2 = params['lw']['pb2'].shape[0]
    p2 = params['c2']
    cwa = p2['pW1'][:h1]
    cwb = params['upW'] @ p2['pW1'][h1:]
    cb1 = p2['pb1'] + params['upb'] @ p2['pW1'][h1:]
    c2_args = (cwa, cwb, _row(cb1), p2['pW2'], _row(p2['pb2']),
               _edge_w4(p2, h1), _row(p2['nnb']), p2['gWih'],
               _row(p2['gbih']), p2['gWhh'], _row(p2['gbhh']))
    gsplit = 2
    th = t // gsplit
    fn = functools.partial(_unet_kern, th, nx, h1, h2)
    nf = t * nx * ny
    nfh = nf // gsplit
    def _wspec(a):
        return pl.BlockSpec(a.shape, lambda g: (0,) * a.ndim)
    call = lambda xx, *ws: pl.pallas_call(
        fn,
        grid=(gsplit,),
        in_specs=[pl.BlockSpec((th, nx, ny, c), lambda g: (g, 0, 0, 0))]
        + [_wspec(w) for w in ws],
        out_specs=pl.BlockSpec((th, nx, ny, h1), lambda g: (g, 0, 0, 0)),
        out_shape=jax.ShapeDtypeStruct((t, nx, ny, h1), _F32),
        scratch_shapes=[pltpu.VMEM((nfh, h1), _F32),
                        pltpu.VMEM((nfh, h1), _F32)],
        compiler_params=pltpu.CompilerParams(
            dimension_semantics=("parallel",)),
    )(xx, *ws)
    outs = []
    for bi in range(b):
        h3 = call(inputs[bi], *_block_args(params['c1'], h1),
                  *_block_args(params['lw'], h2), *c2_args)
        outs.append(h3)
    return jnp.stack(outs, 0)


# speedup vs baseline: 1.2769x; 1.2721x over previous
import functools

import jax
import jax.numpy as jnp
from jax.experimental import pallas as pl
from jax.experimental.pallas import tpu as pltpu

_F32 = jnp.float32


def _dot(a, b):
    return jnp.dot(a, b, preferred_element_type=_F32)


def _w4_build(ew1, eb1, ew2, eb2, h, w4s_ref):
    r0 = ew1[0:1, :]
    r1 = ew1[1:2, :]
    act = jnp.maximum(jnp.concatenate([r0, -r0, r1, -r1], axis=0) + eb1, 0.0)
    wf = _dot(act, ew2) + eb2
    for i in range(h):
        w4s_ref[pl.Slice(i, 4, h), :] = wf[:, i * h:(i + 1) * h]
    return w4s_ref[...]


def _gru_core(nx, h, hid, w4, nnb, gwih, gbih, gwhh, gbhh):
    n = hid.shape[0]
    row = jax.lax.broadcasted_iota(jnp.int32, (n, 1), 0)
    j = row % nx
    i = (row // nx) % nx
    m0 = (j >= 1)
    m1 = (j <= nx - 2)
    m2 = (i >= 1)
    m3 = (i <= nx - 2)
    z1 = jnp.zeros((1, h), _F32)
    znx = jnp.zeros((nx, h), _F32)
    s0 = jnp.where(m0, jnp.concatenate([z1, hid[:-1]], axis=0), 0.0)
    s1 = jnp.where(m1, jnp.concatenate([hid[1:], z1], axis=0), 0.0)
    s2 = jnp.where(m2, jnp.concatenate([znx, hid[:-nx]], axis=0), 0.0)
    s3 = jnp.where(m3, jnp.concatenate([hid[nx:], znx], axis=0), 0.0)
    xcat = jnp.concatenate([s0, s1, s2, s3], axis=1)
    ssum = _dot(xcat, w4)
    deg = (m0.astype(_F32) + m1.astype(_F32) + m2.astype(_F32)
           + m3.astype(_F32))
    m = jnp.maximum(ssum * (1.0 / deg) + nnb, 0.0)
    gi = _dot(m, gwih) + gbih
    gh = _dot(hid, gwhh) + gbhh
    r = jax.nn.sigmoid(gi[:, :h] + gh[:, :h])
    z = jax.nn.sigmoid(gi[:, h:2 * h] + gh[:, h:2 * h])
    nn = jnp.tanh(gi[:, 2 * h:] + r * gh[:, 2 * h:])
    return (1.0 - z) * nn + z * hid


def _unet_kern(t, nx, h1, h2,
               x_ref,
               aw1, ab1, aw2, ab2, ae1, aeb1, ae2, aeb2, anb,
               awih, abih, awhh, abhh,
               bw1, bb1, bw2, bb2, be1, beb1, be2, beb2, bnb,
               bwih, bbih, bwhh, bbhh,
               cw1, cb1r, cw2, cb2, ce1, ceb1, ce2, ceb2, cnb,
               cwih, cbih, cwhh, cbhh,
               upw, upb,
               o_ref, spool_ref, sup_ref, w4a_ref, w4b_ref, w4c_ref):
    nf = t * nx * nx
    nh = nx // 2
    nc = t * nh * nh

    a4 = _w4_build(ae1[...], aeb1[...], ae2[...], aeb2[...], h1, w4a_ref)
    b4 = _w4_build(be1[...], beb1[...], be2[...], beb2[...], h2, w4b_ref)
    c4 = _w4_build(ce1[...], ceb1[...], ce2[...], ceb2[...], h1, w4c_ref)

    x = x_ref[...].reshape(nf, x_ref.shape[-1])
    l1 = jnp.maximum(_dot(x, aw1[...]) + ab1[...], 0.0)
    hid = _dot(l1, aw2[...]) + ab2[...]
    bp = _gru_core(nx, h1, hid, a4, anb[...], awih[...],
                   abih[...], awhh[...], abhh[...])

    z1 = jnp.zeros((1, h1), _F32)
    spool_ref[...] = bp + jnp.concatenate([bp[1:], z1], axis=0)
    t1 = spool_ref[pl.Slice(0, nf // 2, 2), :]
    t4 = t1.reshape(t * nx // 2, 2, nh, h1)
    d = ((t4[:, 0] + t4[:, 1]) * 0.25).reshape(nc, h1)

    l1b = jnp.maximum(_dot(d, bw1[...]) + bb1[...], 0.0)
    hidb = _dot(l1b, bw2[...]) + bb2[...]
    h2v = _gru_core(nh, h2, hidb, b4, bnb[...],
                    bwih[...], bbih[...], bwhh[...], bbhh[...])

    cpw1 = cw1[...]
    cwa = cpw1[:h1]
    cwb = _dot(upw[...], cpw1[h1:])
    cb1 = cb1r[...] + _dot(upb[...], cpw1[h1:])

    u3 = h2v.reshape(t * nh, 1, nh, h2)
    ui = jnp.concatenate([u3, u3], axis=1).reshape(nf // 2, h2)
    v = _dot(ui, cwb)
    sup_ref[pl.Slice(0, nf // 2, 2), :] = v
    sup_ref[pl.Slice(1, nf // 2, 2), :] = v

    pre = _dot(bp, cwa) + sup_ref[...] + cb1
    l1c = jnp.maximum(pre, 0.0)
    hidc = _dot(l1c, cw2[...]) + cb2[...]
    res = _gru_core(nx, h1, hidc, c4, cnb[...],
                    cwih[...], cbih[...], cwhh[...], cbhh[...])
    o_ref[...] = res.reshape(o_ref.shape)


def _row(v):
    return v.reshape(1, -1)


def _block_args(p):
    return (p['pW1'], _row(p['pb1']), p['pW2'], _row(p['pb2']),
            p['eW1'], _row(p['eb1']), p['eW2'], _row(p['eb2']),
            _row(p['nnb']), p['gWih'], _row(p['gbih']), p['gWhh'],
            _row(p['gbhh']))


def kernel(inputs, params):
    b, t, nx, ny, c = inputs.shape
    h1 = params['c1']['pb2'].shape[0]
    h2 = params['lw']['pb2'].shape[0]
    fn = functools.partial(_unet_kern, t, nx, h1, h2)
    nf = t * nx * ny

    def _wspec(a):
        return pl.BlockSpec(a.shape, lambda *_: (0,) * a.ndim)

    call = lambda xx, *ws: pl.pallas_call(
        fn,
        out_shape=jax.ShapeDtypeStruct((t, nx, ny, h1), _F32),
        scratch_shapes=[pltpu.VMEM((nf, h1), _F32),
                        pltpu.VMEM((nf, h1), _F32),
                        pltpu.VMEM((4 * h1, h1), _F32),
                        pltpu.VMEM((4 * h2, h2), _F32),
                        pltpu.VMEM((4 * h1, h1), _F32)],
    )(xx, *ws)
    outs = []
    for bi in range(b):
        h3 = call(inputs[bi], *_block_args(params['c1']),
                  *_block_args(params['lw']), *_block_args(params['c2']),
                  params['upW'], _row(params['upb']))
        outs.append(h3)
    return jnp.stack(outs, 0)


# speedup vs baseline: 1.5646x; 1.2253x over previous
import functools

import jax
import jax.numpy as jnp
from jax.experimental import pallas as pl
from jax.experimental.pallas import tpu as pltpu

_F32 = jnp.float32


def _dot(a, b):
    return jnp.dot(a, b, preferred_element_type=_F32)


def _rv(ref):
    return ref[...].reshape(1, -1)


def _dot_rt(a, bt):
    return jax.lax.dot_general(a, bt, (((1,), (1,)), ((), ())),
                               preferred_element_type=_F32)


def _w4_build(ew1, eb1, ew2, eb2, h, w4s_ref):
    r0 = ew1[0:1, :]
    r1 = ew1[1:2, :]
    act = jnp.maximum(jnp.concatenate([r0, -r0, r1, -r1], axis=0) + eb1, 0.0)
    wf = _dot(act, ew2) + eb2
    for i in range(h):
        w4s_ref[pl.Slice(i, 4, h), :] = wf[:, i * h:(i + 1) * h]
    return w4s_ref[...]


def _gru_core(nx, h, hid, w4, nnb, gwih, gbih, gwhh, gbhh):
    n = hid.shape[0]
    row = jax.lax.broadcasted_iota(jnp.int32, (n, 1), 0)
    j = row % nx
    i = (row // nx) % nx
    m0 = (j >= 1)
    m1 = (j <= nx - 2)
    m2 = (i >= 1)
    m3 = (i <= nx - 2)
    z1 = jnp.zeros((1, h), _F32)
    znx = jnp.zeros((nx, h), _F32)
    s0 = jnp.where(m0, jnp.concatenate([z1, hid[:-1]], axis=0), 0.0)
    s1 = jnp.where(m1, jnp.concatenate([hid[1:], z1], axis=0), 0.0)
    s2 = jnp.where(m2, jnp.concatenate([znx, hid[:-nx]], axis=0), 0.0)
    s3 = jnp.where(m3, jnp.concatenate([hid[nx:], znx], axis=0), 0.0)
    xcat = jnp.concatenate([s0, s1, s2, s3], axis=1)
    ssum = _dot(xcat, w4)
    deg = (m0.astype(_F32) + m1.astype(_F32) + m2.astype(_F32)
           + m3.astype(_F32))
    m = jnp.maximum(ssum * (1.0 / deg) + nnb, 0.0)
    gi = _dot(m, gwih) + gbih
    gh = _dot(hid, gwhh) + gbhh
    r = jax.nn.sigmoid(gi[:, :h] + gh[:, :h])
    z = jax.nn.sigmoid(gi[:, h:2 * h] + gh[:, h:2 * h])
    nn = jnp.tanh(gi[:, 2 * h:] + r * gh[:, 2 * h:])
    return (1.0 - z) * nn + z * hid


def _unet_kern(t, nx, h1, h2,
               x_ref,
               aw1, ab1, aw2, ab2, ae1, aeb1, ae2, aeb2, anb,
               awih, abih, awhh, abhh,
               bw1, bb1, bw2, bb2, be1, beb1, be2, beb2, bnb,
               bwih, bbih, bwhh, bbhh,
               cw1t, cb1r, cw2, cb2, ce1, ceb1, ce2, ceb2, cnb,
               cwih, cbih, cwhh, cbhh,
               upwt, upb,
               o_ref, spool_ref, sup_ref, w4a_ref, w4b_ref, w4c_ref):
    nf = t * nx * nx
    nh = nx // 2
    nc = t * nh * nh

    a4 = _w4_build(ae1[...], _rv(aeb1), ae2[...], _rv(aeb2), h1, w4a_ref)
    b4 = _w4_build(be1[...], _rv(beb1), be2[...], _rv(beb2), h2, w4b_ref)
    c4 = _w4_build(ce1[...], _rv(ceb1), ce2[...], _rv(ceb2), h1, w4c_ref)

    x = x_ref[...].reshape(nf, x_ref.shape[-1])
    l1 = jnp.maximum(_dot(x, aw1[...]) + _rv(ab1), 0.0)
    hid = _dot(l1, aw2[...]) + _rv(ab2)
    bp = _gru_core(nx, h1, hid, a4, _rv(anb), awih[...],
                   _rv(abih), awhh[...], _rv(abhh))

    z1 = jnp.zeros((1, h1), _F32)
    spool_ref[...] = bp + jnp.concatenate([bp[1:], z1], axis=0)
    t1 = spool_ref[pl.Slice(0, nf // 2, 2), :]
    t4 = t1.reshape(t * nx // 2, 2, nh, h1)
    d = ((t4[:, 0] + t4[:, 1]) * 0.25).reshape(nc, h1)

    l1b = jnp.maximum(_dot(d, bw1[...]) + _rv(bb1), 0.0)
    hidb = _dot(l1b, bw2[...]) + _rv(bb2)
    h2v = _gru_core(nh, h2, hidb, b4, _rv(bnb),
                    bwih[...], _rv(bbih), bwhh[...], _rv(bbhh))

    cpw1t = cw1t[...]
    topt = cpw1t[:, :h1]
    bott = cpw1t[:, h1:]
    cb1 = _rv(cb1r) + _dot_rt(_rv(upb), bott)

    u3 = h2v.reshape(t * nh, 1, nh, h2)
    ui = jnp.concatenate([u3, u3], axis=1).reshape(nf // 2, h2)
    v = _dot_rt(_dot_rt(ui, upwt[...]), bott)
    sup_ref[pl.Slice(0, nf // 2, 2), :] = v
    sup_ref[pl.Slice(1, nf // 2, 2), :] = v

    pre = _dot_rt(bp, topt) + sup_ref[...] + cb1
    l1c = jnp.maximum(pre, 0.0)
    hidc = _dot(l1c, cw2[...]) + _rv(cb2)
    res = _gru_core(nx, h1, hidc, c4, _rv(cnb),
                    cwih[...], _rv(cbih), cwhh[...], _rv(cbhh))
    o_ref[...] = res.reshape(o_ref.shape)


def _row(v):
    return v.reshape(1, -1)


def _block_args(p):
    return (p['pW1'], p['pb1'], p['pW2'], p['pb2'],
            p['eW1'], p['eb1'], p['eW2'], p['eb2'],
            p['nnb'], p['gWih'], p['gbih'], p['gWhh'],
            p['gbhh'])


def kernel(inputs, params):
    b, t, nx, ny, c = inputs.shape
    h1 = params['c1']['pb2'].shape[0]
    h2 = params['lw']['pb2'].shape[0]
    fn = functools.partial(_unet_kern, t, nx, h1, h2)
    nf = t * nx * ny

    def _wspec(a):
        return pl.BlockSpec(a.shape, lambda *_: (0,) * a.ndim)

    call = lambda xx, *ws: pl.pallas_call(
        fn,
        out_shape=jax.ShapeDtypeStruct((t, nx, ny, h1), _F32),
        scratch_shapes=[pltpu.VMEM((nf, h1), _F32),
                        pltpu.VMEM((nf, h1), _F32),
                        pltpu.VMEM((4 * h1, h1), _F32),
                        pltpu.VMEM((4 * h2, h2), _F32),
                        pltpu.VMEM((4 * h1, h1), _F32)],
    )(xx, *ws)
    outs = []
    for bi in range(b):
        c2a = list(_block_args(params['c2']))
        c2a[0] = params['c2']['pW1'].T
        h3 = call(inputs[bi], *_block_args(params['c1']),
                  *_block_args(params['lw']), *c2a,
                  params['upW'].T, params['upb'])
        outs.append(h3)
    return jnp.stack(outs, 0)
